# Initial kernel scaffold; baseline (speedup 1.0000x reference)
import jax
import jax.numpy as jnp
from jax.experimental import pallas as pl


def kernel(x, pre_bias, latent_bias, W_enc, W_dec):
    raise NotImplementedError("write your pallas kernel here")


# speedup vs baseline: 2.2011x; 2.2011x over previous
import functools

import jax
import jax.numpy as jnp
from jax.experimental import pallas as pl
from jax.experimental.pallas import tpu as pltpu


def _encode_body(x_ref, pb_ref, lb_ref, w_ref, out_ref):
    xc = x_ref[...] - pb_ref[...]
    acc = jax.lax.dot_general(
        xc, w_ref[...], (((1,), (1,)), ((), ())),
        preferred_element_type=jnp.float32)
    out_ref[...] = acc + lb_ref[...]


def _encode(x, pre_bias2d, latent_bias2d, W_enc, h_tile, b_tile):
    B, D = x.shape
    H = W_enc.shape[0]
    nh = H // h_tile
    nb = B // b_tile
    return pl.pallas_call(
        _encode_body,
        grid=(nh, nb),
        in_specs=[
            pl.BlockSpec((b_tile, D), lambda h, b: (b, 0)),
            pl.BlockSpec((1, D), lambda h, b: (0, 0)),
            pl.BlockSpec((1, h_tile), lambda h, b: (0, h)),
            pl.BlockSpec((h_tile, D), lambda h, b: (h, 0)),
        ],
        out_specs=pl.BlockSpec((b_tile, h_tile), lambda h, b: (b, h)),
        out_shape=jax.ShapeDtypeStruct((B, H), jnp.float32),
    )(x, pre_bias2d, latent_bias2d, W_enc)


def _topk_body(pa_ref, sc_ref, tv_ref, ti_ref, work_ref, *, K):
    pa = pa_ref[...]
    r = jnp.maximum(pa, 0.0)
    Bt, H = r.shape
    iota = jax.lax.broadcasted_iota(jnp.int32, (Bt, H), 1)
    kiota = jax.lax.broadcasted_iota(jnp.int32, (Bt, K), 1)
    work_ref[...] = r

    def body(k, carry):
        vals, inds = carry
        work = work_ref[...]
        m = jnp.max(work, axis=1, keepdims=True)
        cand = jnp.where(work == m, iota, H)
        idx = jnp.min(cand, axis=1, keepdims=True)
        work_ref[...] = jnp.where(iota == idx, -1.0, work)
        vals = jnp.where(kiota == k, m, vals)
        inds = jnp.where(kiota == k, idx, inds)
        return vals, inds

    vals0 = jnp.zeros((Bt, K), jnp.float32)
    inds0 = jnp.zeros((Bt, K), jnp.int32)
    vals, inds = jax.lax.fori_loop(0, K, body, (vals0, inds0))
    tv_ref[...] = vals
    ti_ref[...] = inds
    sc_ref[...] = jnp.where(work_ref[...] < 0.0, r, 0.0)


def _topk(pre_acts, K, b_tile):
    B, H = pre_acts.shape
    nb = B // b_tile
    return pl.pallas_call(
        functools.partial(_topk_body, K=K),
        grid=(nb,),
        in_specs=[pl.BlockSpec((b_tile, H), lambda b: (b, 0))],
        out_specs=[
            pl.BlockSpec((b_tile, H), lambda b: (b, 0)),
            pl.BlockSpec((b_tile, K), lambda b: (b, 0)),
            pl.BlockSpec((b_tile, K), lambda b: (b, 0)),
        ],
        out_shape=[
            jax.ShapeDtypeStruct((B, H), jnp.float32),
            jax.ShapeDtypeStruct((B, K), jnp.float32),
            jax.ShapeDtypeStruct((B, K), jnp.int32),
        ],
        scratch_shapes=[pltpu.VMEM((b_tile, H), jnp.float32)],
    )(pre_acts)


def _decode_body(sc_ref, wd_ref, pb_ref, out_ref):
    h = pl.program_id(1)

    @pl.when(h == 0)
    def _():
        out_ref[...] = jnp.broadcast_to(pb_ref[...], out_ref.shape)

    out_ref[...] += jax.lax.dot_general(
        sc_ref[...], wd_ref[...], (((1,), (1,)), ((), ())),
        preferred_element_type=jnp.float32)


def _decode(sparse_code, W_dec, pre_bias2d, h_tile, b_tile):
    B, H = sparse_code.shape
    D = W_dec.shape[0]
    nh = H // h_tile
    nb = B // b_tile
    return pl.pallas_call(
        _decode_body,
        grid=(nb, nh),
        in_specs=[
            pl.BlockSpec((b_tile, h_tile), lambda b, h: (b, h)),
            pl.BlockSpec((D, h_tile), lambda b, h: (0, h)),
            pl.BlockSpec((1, D), lambda b, h: (0, 0)),
        ],
        out_specs=pl.BlockSpec((b_tile, D), lambda b, h: (b, 0)),
        out_shape=jax.ShapeDtypeStruct((B, D), jnp.float32),
    )(sparse_code, W_dec, pre_bias2d)


def kernel(x, pre_bias, latent_bias, W_enc, W_dec):
    B, D = x.shape
    H = W_enc.shape[0]
    K = 50
    pb2 = pre_bias.reshape(1, D)
    lb2 = latent_bias.reshape(1, H)

    pre_acts = _encode(x, pb2, lb2, W_enc, h_tile=2048, b_tile=1024)
    sparse_code, topk_values, topk_indices = _topk(pre_acts, K, b_tile=32)
    reconstruction = _decode(sparse_code, W_dec, pb2, h_tile=1024, b_tile=2048)
    return (reconstruction, sparse_code, pre_acts, topk_values, topk_indices)


# speedup vs baseline: 4.2560x; 1.9335x over previous
import functools

import jax
import jax.numpy as jnp
from jax.experimental import pallas as pl
from jax.experimental.pallas import tpu as pltpu


def _encode_body(x_ref, pb_ref, lb_ref, w_ref, out_ref):
    xc = x_ref[...] - pb_ref[...]
    acc = jax.lax.dot_general(
        xc, w_ref[...], (((1,), (1,)), ((), ())),
        preferred_element_type=jnp.float32)
    out_ref[...] = acc + lb_ref[...]


def _encode(x, pre_bias2d, latent_bias2d, W_enc, h_tile, b_tile):
    B, D = x.shape
    H = W_enc.shape[0]
    nh = H // h_tile
    nb = B // b_tile
    return pl.pallas_call(
        _encode_body,
        grid=(nh, nb),
        in_specs=[
            pl.BlockSpec((b_tile, D), lambda h, b: (b, 0)),
            pl.BlockSpec((1, D), lambda h, b: (0, 0)),
            pl.BlockSpec((1, h_tile), lambda h, b: (0, h)),
            pl.BlockSpec((h_tile, D), lambda h, b: (h, 0)),
        ],
        out_specs=pl.BlockSpec((b_tile, h_tile), lambda h, b: (b, h)),
        out_shape=jax.ShapeDtypeStruct((B, H), jnp.float32),
    )(x, pre_bias2d, latent_bias2d, W_enc)


def _topk_body(pa_ref, sc_ref, tv_ref, ti_ref, work_ref, *, K, T):
    pa = pa_ref[...]
    r = jnp.maximum(pa, 0.0)
    Bt, H = r.shape
    C = min(128, H)
    S = H // C
    r3 = r.reshape(Bt, S, C)
    s_iota = jax.lax.broadcasted_iota(jnp.int32, (Bt, S, C), 1)
    kiota = jax.lax.broadcasted_iota(jnp.int32, (Bt, K), 1)

    work_ref[...] = r

    def chunk_body(t, carry):
        cv, ci = carry
        w3 = work_ref[...].reshape(Bt, S, C)
        mt = jnp.max(w3, axis=1, keepdims=True)
        st = jnp.min(jnp.where(w3 == mt, s_iota, S), axis=1,
                     keepdims=True)
        work_ref[...] = jnp.where(s_iota == st, -1.0, w3).reshape(Bt, H)
        tio = jax.lax.broadcasted_iota(jnp.int32, (Bt, T, C), 1)
        cv = jnp.where(tio == t, mt, cv)
        gidx = st * C + jax.lax.broadcasted_iota(jnp.int32, (Bt, 1, C), 2)
        ci = jnp.where(tio == t, gidx, ci)
        return cv, ci

    cv0 = jnp.full((Bt, T, C), -1.0, jnp.float32)
    ci0 = jnp.full((Bt, T, C), H, jnp.int32)
    cand_v, cand_i = jax.lax.fori_loop(0, T, chunk_body, (cv0, ci0))

    def ext_body(k, carry):
        cv, ci, vals, inds = carry
        m = jnp.max(cv, axis=(1, 2), keepdims=True)
        i = jnp.min(jnp.where(cv == m, ci, H), axis=(1, 2),
                    keepdims=True)
        cv = jnp.where(ci == i, -1.0, cv)
        vals = jnp.where(kiota == k, m[:, :, 0], vals)
        inds = jnp.where(kiota == k, i[:, :, 0], inds)
        return cv, ci, vals, inds

    vals0 = jnp.zeros((Bt, K), jnp.float32)
    inds0 = jnp.zeros((Bt, K), jnp.int32)
    cand_left, _, vals, inds = jax.lax.fori_loop(
        0, K, ext_body, (cand_v, cand_i, vals0, inds0))
    tv_ref[...] = vals
    ti_ref[...] = inds

    mK = vals[:, K - 1][:, None]
    iK = inds[:, K - 1][:, None]
    iota = jax.lax.broadcasted_iota(jnp.int32, (Bt, H), 1)
    sel = (r > mK) | ((r == mK) & (iota <= iK))
    sc_ref[...] = jnp.where(sel, r, 0.0)

    chunk_used = jnp.sum(jnp.where(cand_left < 0.0, 1, 0), axis=1)
    saturated = jnp.max(chunk_used, axis=(0, 1)) >= T
    dup = jnp.max(jnp.where(vals[:, :-1] == vals[:, 1:], 1, 0), axis=(0, 1)) > 0
    zero_hit = jnp.min(vals, axis=(0, 1)) <= 0.0
    flag = saturated | dup | zero_hit

    @pl.when(flag)
    def _slow():
        work_ref[...] = r

        def body(k, carry):
            svals, sinds = carry
            work = work_ref[...]
            m = jnp.max(work, axis=1, keepdims=True)
            cand = jnp.where(work == m, iota, H)
            idx = jnp.min(cand, axis=1, keepdims=True)
            work_ref[...] = jnp.where(iota == idx, -1.0, work)
            svals = jnp.where(kiota == k, m, svals)
            sinds = jnp.where(kiota == k, idx, sinds)
            return svals, sinds

        svals, sinds = jax.lax.fori_loop(0, K, body, (vals0, inds0))
        tv_ref[...] = svals
        ti_ref[...] = sinds
        sc_ref[...] = jnp.where(work_ref[...] < 0.0, r, 0.0)


def _topk(pre_acts, K, b_tile, T=8):
    B, H = pre_acts.shape
    nb = B // b_tile
    return pl.pallas_call(
        functools.partial(_topk_body, K=K, T=T),
        grid=(nb,),
        in_specs=[pl.BlockSpec((b_tile, H), lambda b: (b, 0))],
        out_specs=[
            pl.BlockSpec((b_tile, H), lambda b: (b, 0)),
            pl.BlockSpec((b_tile, K), lambda b: (b, 0)),
            pl.BlockSpec((b_tile, K), lambda b: (b, 0)),
        ],
        out_shape=[
            jax.ShapeDtypeStruct((B, H), jnp.float32),
            jax.ShapeDtypeStruct((B, K), jnp.float32),
            jax.ShapeDtypeStruct((B, K), jnp.int32),
        ],
        scratch_shapes=[pltpu.VMEM((b_tile, H), jnp.float32)],
    )(pre_acts)


def _decode_body(sc_ref, wd_ref, pb_ref, out_ref):
    h = pl.program_id(1)

    @pl.when(h == 0)
    def _():
        out_ref[...] = jnp.broadcast_to(pb_ref[...], out_ref.shape)

    out_ref[...] += jax.lax.dot_general(
        sc_ref[...], wd_ref[...], (((1,), (1,)), ((), ())),
        preferred_element_type=jnp.float32)


def _decode(sparse_code, W_dec, pre_bias2d, h_tile, b_tile):
    B, H = sparse_code.shape
    D = W_dec.shape[0]
    nh = H // h_tile
    nb = B // b_tile
    return pl.pallas_call(
        _decode_body,
        grid=(nb, nh),
        in_specs=[
            pl.BlockSpec((b_tile, h_tile), lambda b, h: (b, h)),
            pl.BlockSpec((D, h_tile), lambda b, h: (0, h)),
            pl.BlockSpec((1, D), lambda b, h: (0, 0)),
        ],
        out_specs=pl.BlockSpec((b_tile, D), lambda b, h: (b, 0)),
        out_shape=jax.ShapeDtypeStruct((B, D), jnp.float32),
    )(sparse_code, W_dec, pre_bias2d)


def kernel(x, pre_bias, latent_bias, W_enc, W_dec):
    B, D = x.shape
    H = W_enc.shape[0]
    K = 50
    pb2 = pre_bias.reshape(1, D)
    lb2 = latent_bias.reshape(1, H)

    pre_acts = _encode(x, pb2, lb2, W_enc, h_tile=2048, b_tile=1024)
    sparse_code, topk_values, topk_indices = _topk(pre_acts, K, b_tile=32)
    reconstruction = _decode(sparse_code, W_dec, pb2, h_tile=1024, b_tile=2048)
    return (reconstruction, sparse_code, pre_acts, topk_values, topk_indices)


# speedup vs baseline: 4.2569x; 1.0002x over previous
import functools

import jax
import jax.numpy as jnp
from jax.experimental import pallas as pl
from jax.experimental.pallas import tpu as pltpu


def _encode_body(x_ref, pb_ref, lb_ref, w_ref, out_ref):
    xc = x_ref[...] - pb_ref[...]
    acc = jax.lax.dot_general(
        xc, w_ref[...], (((1,), (1,)), ((), ())),
        preferred_element_type=jnp.float32)
    out_ref[...] = acc + lb_ref[...]


def _encode(x, pre_bias2d, latent_bias2d, W_enc, h_tile, b_tile):
    B, D = x.shape
    H = W_enc.shape[0]
    nh = H // h_tile
    nb = B // b_tile
    return pl.pallas_call(
        _encode_body,
        grid=(nh, nb),
        in_specs=[
            pl.BlockSpec((b_tile, D), lambda h, b: (b, 0)),
            pl.BlockSpec((1, D), lambda h, b: (0, 0)),
            pl.BlockSpec((1, h_tile), lambda h, b: (0, h)),
            pl.BlockSpec((h_tile, D), lambda h, b: (h, 0)),
        ],
        out_specs=pl.BlockSpec((b_tile, h_tile), lambda h, b: (b, h)),
        out_shape=jax.ShapeDtypeStruct((B, H), jnp.float32),
    )(x, pre_bias2d, latent_bias2d, W_enc)


def _topk_body(pa_ref, sc_ref, tv_ref, ti_ref, work_ref, *, K, T):
    pa = pa_ref[...]
    r = jnp.maximum(pa, 0.0)
    Bt, H = r.shape
    C = min(128, H)
    S = H // C
    r3 = r.reshape(Bt, S, C)
    s_iota = jax.lax.broadcasted_iota(jnp.int32, (Bt, S, C), 1)
    kiota = jax.lax.broadcasted_iota(jnp.int32, (Bt, K), 1)

    work_ref[...] = r

    def chunk_body(t, carry):
        cv, ci = carry
        w3 = work_ref[...].reshape(Bt, S, C)
        mt = jnp.max(w3, axis=1, keepdims=True)
        st = jnp.min(jnp.where(w3 == mt, s_iota, S), axis=1,
                     keepdims=True)
        work_ref[...] = jnp.where(s_iota == st, -1.0, w3).reshape(Bt, H)
        tio = jax.lax.broadcasted_iota(jnp.int32, (Bt, T, C), 1)
        cv = jnp.where(tio == t, mt, cv)
        gidx = st * C + jax.lax.broadcasted_iota(jnp.int32, (Bt, 1, C), 2)
        ci = jnp.where(tio == t, gidx, ci)
        return cv, ci

    cv0 = jnp.full((Bt, T, C), -1.0, jnp.float32)
    ci0 = jnp.full((Bt, T, C), H, jnp.int32)
    cand_v, cand_i = jax.lax.fori_loop(0, T, chunk_body, (cv0, ci0))

    def ext_body(k, carry):
        cv, ci, vals, inds = carry
        m = jnp.max(cv, axis=(1, 2), keepdims=True)
        i = jnp.min(jnp.where(cv == m, ci, H), axis=(1, 2),
                    keepdims=True)
        cv = jnp.where(ci == i, -1.0, cv)
        vals = jnp.where(kiota == k, m[:, :, 0], vals)
        inds = jnp.where(kiota == k, i[:, :, 0], inds)
        return cv, ci, vals, inds

    vals0 = jnp.zeros((Bt, K), jnp.float32)
    inds0 = jnp.zeros((Bt, K), jnp.int32)
    cand_left, _, vals, inds = jax.lax.fori_loop(
        0, K, ext_body, (cand_v, cand_i, vals0, inds0))
    tv_ref[...] = vals
    ti_ref[...] = inds

    mK = vals[:, K - 1][:, None]
    iK = inds[:, K - 1][:, None]
    iota = jax.lax.broadcasted_iota(jnp.int32, (Bt, H), 1)
    sel = (r > mK) | ((r == mK) & (iota <= iK))
    sc_ref[...] = jnp.where(sel, r, 0.0)

    chunk_used = jnp.sum(jnp.where(cand_left < 0.0, 1, 0), axis=1)
    saturated = jnp.max(chunk_used, axis=(0, 1)) >= T
    dup = jnp.max(jnp.where(vals[:, :-1] == vals[:, 1:], 1, 0), axis=(0, 1)) > 0
    zero_hit = jnp.min(vals, axis=(0, 1)) <= 0.0
    flag = saturated | dup | zero_hit

    @pl.when(flag)
    def _slow():
        work_ref[...] = r

        def body(k, carry):
            svals, sinds = carry
            work = work_ref[...]
            m = jnp.max(work, axis=1, keepdims=True)
            cand = jnp.where(work == m, iota, H)
            idx = jnp.min(cand, axis=1, keepdims=True)
            work_ref[...] = jnp.where(iota == idx, -1.0, work)
            svals = jnp.where(kiota == k, m, svals)
            sinds = jnp.where(kiota == k, idx, sinds)
            return svals, sinds

        svals, sinds = jax.lax.fori_loop(0, K, body, (vals0, inds0))
        tv_ref[...] = svals
        ti_ref[...] = sinds
        sc_ref[...] = jnp.where(work_ref[...] < 0.0, r, 0.0)


def _topk(pre_acts, K, b_tile, T=8):
    B, H = pre_acts.shape
    nb = B // b_tile
    return pl.pallas_call(
        functools.partial(_topk_body, K=K, T=T),
        grid=(nb,),
        in_specs=[pl.BlockSpec((b_tile, H), lambda b: (b, 0))],
        out_specs=[
            pl.BlockSpec((b_tile, H), lambda b: (b, 0)),
            pl.BlockSpec((b_tile, K), lambda b: (b, 0)),
            pl.BlockSpec((b_tile, K), lambda b: (b, 0)),
        ],
        out_shape=[
            jax.ShapeDtypeStruct((B, H), jnp.float32),
            jax.ShapeDtypeStruct((B, K), jnp.float32),
            jax.ShapeDtypeStruct((B, K), jnp.int32),
        ],
        scratch_shapes=[pltpu.VMEM((b_tile, H), jnp.float32)],
    )(pre_acts)


def _decode_body(sc_ref, wd_ref, pb_ref, out_ref):
    h = pl.program_id(1)

    @pl.when(h == 0)
    def _():
        out_ref[...] = jnp.broadcast_to(pb_ref[...], out_ref.shape)

    out_ref[...] += jax.lax.dot_general(
        sc_ref[...].astype(jnp.bfloat16), wd_ref[...].astype(jnp.bfloat16),
        (((1,), (1,)), ((), ())),
        preferred_element_type=jnp.float32)


def _decode(sparse_code, W_dec, pre_bias2d, h_tile, b_tile):
    B, H = sparse_code.shape
    D = W_dec.shape[0]
    nh = H // h_tile
    nb = B // b_tile
    return pl.pallas_call(
        _decode_body,
        grid=(nb, nh),
        in_specs=[
            pl.BlockSpec((b_tile, h_tile), lambda b, h: (b, h)),
            pl.BlockSpec((D, h_tile), lambda b, h: (0, h)),
            pl.BlockSpec((1, D), lambda b, h: (0, 0)),
        ],
        out_specs=pl.BlockSpec((b_tile, D), lambda b, h: (b, 0)),
        out_shape=jax.ShapeDtypeStruct((B, D), jnp.float32),
    )(sparse_code, W_dec, pre_bias2d)


def kernel(x, pre_bias, latent_bias, W_enc, W_dec):
    B, D = x.shape
    H = W_enc.shape[0]
    K = 50
    pb2 = pre_bias.reshape(1, D)
    lb2 = latent_bias.reshape(1, H)

    pre_acts = _encode(x, pb2, lb2, W_enc, h_tile=2048, b_tile=1024)
    sparse_code, topk_values, topk_indices = _topk(pre_acts, K, b_tile=32)
    reconstruction = _decode(sparse_code, W_dec, pb2, h_tile=1024, b_tile=2048)
    return (reconstruction, sparse_code, pre_acts, topk_values, topk_indices)


# speedup vs baseline: 6.2820x; 1.4757x over previous
import functools

import jax
import jax.numpy as jnp
from jax.experimental import pallas as pl
from jax.experimental.pallas import tpu as pltpu


def _encode_body(x_ref, pb_ref, lb_ref, w_ref, out_ref):
    xc = x_ref[...] - pb_ref[...]
    acc = jax.lax.dot_general(
        xc, w_ref[...], (((1,), (1,)), ((), ())),
        preferred_element_type=jnp.float32)
    out_ref[...] = acc + lb_ref[...]


def _encode(x, pre_bias2d, latent_bias2d, W_enc, h_tile, b_tile):
    B, D = x.shape
    H = W_enc.shape[0]
    nh = H // h_tile
    nb = B // b_tile
    return pl.pallas_call(
        _encode_body,
        grid=(nh, nb),
        in_specs=[
            pl.BlockSpec((b_tile, D), lambda h, b: (b, 0)),
            pl.BlockSpec((1, D), lambda h, b: (0, 0)),
            pl.BlockSpec((1, h_tile), lambda h, b: (0, h)),
            pl.BlockSpec((h_tile, D), lambda h, b: (h, 0)),
        ],
        out_specs=pl.BlockSpec((b_tile, h_tile), lambda h, b: (b, h)),
        out_shape=jax.ShapeDtypeStruct((B, H), jnp.float32),
    )(x, pre_bias2d, latent_bias2d, W_enc)


def _topk_body(pa_ref, sc_ref, tv_ref, ti_ref, work_ref, cv_ref, cg_ref, *, K, T):
    Bt, H = pa_ref.shape
    C = min(128, H)
    S = H // C
    kiota = jax.lax.broadcasted_iota(jnp.int32, (Bt, K), 1)
    NEG = jnp.float32(-jnp.inf)

    mreg = [jnp.full((Bt, C), NEG, jnp.float32) for _ in range(T)]
    sreg = [jnp.full((Bt, C), S, jnp.int32) for _ in range(T)]
    for s in range(S):
        v = pa_ref[:, s * C:(s + 1) * C]
        ci = jnp.full((Bt, C), s, jnp.int32)
        for j in range(T):
            beat = v > mreg[j]
            mo, so = mreg[j], sreg[j]
            mreg[j] = jnp.where(beat, v, mo)
            sreg[j] = jnp.where(beat, ci, so)
            v = jnp.where(beat, mo, v)
            ci = jnp.where(beat, so, ci)
    lane = jax.lax.broadcasted_iota(jnp.int32, (Bt, C), 1)
    for j in range(T):
        cv_ref[:, j, :] = mreg[j]
        cg_ref[:, j, :] = sreg[j] * C + lane

    def ext_body(k, carry):
        vals, inds = carry
        cv = cv_ref[...]
        cg = cg_ref[...]
        m = jnp.max(cv, axis=(1, 2), keepdims=True)
        i = jnp.min(jnp.where(cv == m, cg, H), axis=(1, 2),
                    keepdims=True)
        cv_ref[...] = jnp.where(cg == i, NEG, cv)
        vals = jnp.where(kiota == k, m[:, :, 0], vals)
        inds = jnp.where(kiota == k, i[:, :, 0], inds)
        return vals, inds

    vals0 = jnp.zeros((Bt, K), jnp.float32)
    inds0 = jnp.zeros((Bt, K), jnp.int32)
    vals, inds = jax.lax.fori_loop(0, K, ext_body, (vals0, inds0))
    tv_ref[...] = vals
    ti_ref[...] = inds

    pa = pa_ref[...]
    r = jnp.maximum(pa, 0.0)
    mK = vals[:, K - 1][:, None]
    iK = inds[:, K - 1][:, None]
    iota = jax.lax.broadcasted_iota(jnp.int32, (Bt, H), 1)
    sel = (pa > mK) | ((pa == mK) & (iota <= iK))
    sc_ref[...] = jnp.where(sel, pa, 0.0)

    cand_left = cv_ref[...]
    chunk_used = jnp.sum(jnp.where(cand_left == NEG, 1, 0), axis=1)
    saturated = jnp.max(chunk_used, axis=(0, 1)) >= T
    dup = jnp.max(jnp.where(vals[:, :-1] == vals[:, 1:], 1, 0), axis=(0, 1)) > 0
    nonpos = jnp.min(vals, axis=(0, 1)) <= 0.0
    mrem = jnp.max(cand_left, axis=(1, 2))
    boundary = jnp.max(jnp.where(mrem == vals[:, K - 1], 1, 0), axis=0) > 0
    flag = saturated | dup | nonpos | boundary

    @pl.when(flag)
    def _slow():
        work_ref[...] = r

        def body(k, carry):
            svals, sinds = carry
            work = work_ref[...]
            m = jnp.max(work, axis=1, keepdims=True)
            cand = jnp.where(work == m, iota, H)
            idx = jnp.min(cand, axis=1, keepdims=True)
            work_ref[...] = jnp.where(iota == idx, -1.0, work)
            svals = jnp.where(kiota == k, m, svals)
            sinds = jnp.where(kiota == k, idx, sinds)
            return svals, sinds

        svals, sinds = jax.lax.fori_loop(0, K, body, (vals0, inds0))
        tv_ref[...] = svals
        ti_ref[...] = sinds
        sc_ref[...] = jnp.where(work_ref[...] < 0.0, r, 0.0)


def _topk(pre_acts, K, b_tile, T=8):
    B, H = pre_acts.shape
    nb = B // b_tile
    return pl.pallas_call(
        functools.partial(_topk_body, K=K, T=T),
        grid=(nb,),
        in_specs=[pl.BlockSpec((b_tile, H), lambda b: (b, 0))],
        out_specs=[
            pl.BlockSpec((b_tile, H), lambda b: (b, 0)),
            pl.BlockSpec((b_tile, K), lambda b: (b, 0)),
            pl.BlockSpec((b_tile, K), lambda b: (b, 0)),
        ],
        out_shape=[
            jax.ShapeDtypeStruct((B, H), jnp.float32),
            jax.ShapeDtypeStruct((B, K), jnp.float32),
            jax.ShapeDtypeStruct((B, K), jnp.int32),
        ],
        scratch_shapes=[
            pltpu.VMEM((b_tile, H), jnp.float32),
            pltpu.VMEM((b_tile, T, min(128, H)), jnp.float32),
            pltpu.VMEM((b_tile, T, min(128, H)), jnp.int32),
        ],
    )(pre_acts)


def _decode_body(sc_ref, wd_ref, pb_ref, out_ref):
    h = pl.program_id(1)

    @pl.when(h == 0)
    def _():
        out_ref[...] = jnp.broadcast_to(pb_ref[...], out_ref.shape)

    out_ref[...] += jax.lax.dot_general(
        sc_ref[...].astype(jnp.bfloat16), wd_ref[...].astype(jnp.bfloat16),
        (((1,), (1,)), ((), ())),
        preferred_element_type=jnp.float32)


def _decode(sparse_code, W_dec, pre_bias2d, h_tile, b_tile):
    B, H = sparse_code.shape
    D = W_dec.shape[0]
    nh = H // h_tile
    nb = B // b_tile
    return pl.pallas_call(
        _decode_body,
        grid=(nb, nh),
        in_specs=[
            pl.BlockSpec((b_tile, h_tile), lambda b, h: (b, h)),
            pl.BlockSpec((D, h_tile), lambda b, h: (0, h)),
            pl.BlockSpec((1, D), lambda b, h: (0, 0)),
        ],
        out_specs=pl.BlockSpec((b_tile, D), lambda b, h: (b, 0)),
        out_shape=jax.ShapeDtypeStruct((B, D), jnp.float32),
    )(sparse_code, W_dec, pre_bias2d)


def kernel(x, pre_bias, latent_bias, W_enc, W_dec):
    B, D = x.shape
    H = W_enc.shape[0]
    K = 50
    pb2 = pre_bias.reshape(1, D)
    lb2 = latent_bias.reshape(1, H)

    pre_acts = _encode(x, pb2, lb2, W_enc, h_tile=2048, b_tile=1024)
    sparse_code, topk_values, topk_indices = _topk(pre_acts, K, b_tile=32)
    reconstruction = _decode(sparse_code, W_dec, pb2, h_tile=1024, b_tile=2048)
    return (reconstruction, sparse_code, pre_acts, topk_values, topk_indices)
